# Initial kernel scaffold; baseline (speedup 1.0000x reference)
import jax
import jax.numpy as jnp
from jax.experimental import pallas as pl


def kernel(edge_index, feats, Wself0, Wneigh0, b0, Wself1, Wneigh1, b1, Wself2, Wneigh2, b2):
    raise NotImplementedError("write your pallas kernel here")


# speedup vs baseline: 3.5077x; 3.5077x over previous
import functools

import jax
import jax.numpy as jnp
from jax import lax
from jax.experimental import pallas as pl
from jax.experimental.pallas import tpu as pltpu
from jax.experimental.pallas import tpu_sc as plsc

N = 10000
E = 320000
D = 128

NC = 2
NS = 16
NW = NC * NS
CH_A = 32
CHUNKS_A = 320
SUB_A = 32
NBUF = 8
PRE = NBUF - 1
C0 = 512
C1 = 2 * CHUNKS_A - C0
TOT_CHUNKS = NS * (C0 + C1)
CH_D = 128
CHUNKS_D = 80
SUB_D = 16
NSUB_D = CHUNKS_D // SUB_D
E_PAD = NW * CHUNKS_A * CH_A
N_PAD = 10112
RPT = N_PAD // NS

_mesh = None


def _get_mesh():
    global _mesh
    if _mesh is None:
        _mesh = plsc.VectorSubcoreMesh(
            core_axis_name="c", subcore_axis_name="s",
            num_cores=NC, num_subcores=NS,
        )
    return _mesh

def _wchunks(step):
    out, off = [], 0
    while off < RPT:
        out.append((off, min(step, RPT - off)))
        off += step
    return out


def _sc_agg_body(h_hbm, srcg, dstg, z128, agg_out,
                 src_v, dst_v, rows_v, agg_sh, gsems):
    c = lax.axis_index("c")
    s = lax.axis_index("s")
    base = lax.select(c == 0, s * C0, NS * C0 + s * C1)
    trips = lax.select(c == 0, C0 // SUB_A, C1 // SUB_A)

    pltpu.sync_copy(z128, rows_v.at[0])
    for t, sz in _wchunks(CH_A):
        pltpu.sync_copy(rows_v.at[0, pl.ds(0, sz)],
                        agg_sh.at[pl.ds(s * RPT + t, sz)])
    plsc.subcore_barrier()

    def outer(k, carry):
        pltpu.sync_copy(srcg.at[pl.ds(base + k * SUB_A, SUB_A)], src_v)
        pltpu.sync_copy(dstg.at[pl.ds(base + k * SUB_A, SUB_A)], dst_v)

        descs = [None] * NBUF
        for j in range(SUB_A + PRE):
            if j < SUB_A:
                b = j % NBUF
                descs[b] = pltpu.async_copy(
                    h_hbm.at[src_v.at[j]], rows_v.at[b], gsems.at[b])
            if j >= PRE:
                jj = j - PRE
                b = jj % NBUF
                descs[b].wait()
                pltpu.sync_copy(rows_v.at[b], agg_sh.at[dst_v.at[jj]],
                                add=True)
        return carry

    lax.fori_loop(0, trips, outer, 0)
    plsc.subcore_barrier()

    for t, sz in _wchunks(CH_A):
        sl = pl.ds(s * RPT + t, sz)
        pltpu.sync_copy(agg_sh.at[sl], rows_v.at[0, pl.ds(0, sz)])
        pltpu.sync_copy(rows_v.at[0, pl.ds(0, sz)], agg_out.at[c, sl])


def _sc_deg_body(dstg, z128, ones128, deg_out,
                 dst_v, rows_v, ones_v, deg_sh, gsem):
    c = lax.axis_index("c")
    s = lax.axis_index("s")
    wid = s * NC + c

    pltpu.sync_copy(z128, rows_v)
    pltpu.sync_copy(ones128, ones_v)
    for t, sz in _wchunks(CH_D):
        pltpu.sync_copy(rows_v.at[pl.ds(0, sz)],
                        deg_sh.at[pl.ds(s * RPT + t, sz)])
    plsc.subcore_barrier()

    def outer(k, carry):
        pltpu.sync_copy(dstg.at[wid, pl.ds(k * SUB_D, SUB_D)], dst_v)
        for j in range(SUB_D):
            pltpu.sync_copy(ones_v, deg_sh.at[dst_v.at[j]], add=True)
        return carry

    lax.fori_loop(0, NSUB_D, outer, 0)
    plsc.subcore_barrier()

    for t, sz in _wchunks(CH_D):
        sl = pl.ds(s * RPT + t, sz)
        pltpu.sync_copy(deg_sh.at[sl], rows_v.at[pl.ds(0, sz)])
        pltpu.sync_copy(rows_v.at[pl.ds(0, sz)], deg_out.at[c, sl])


@functools.lru_cache(maxsize=None)
def _get_sc_agg():
    return pl.kernel(
        _sc_agg_body,
        out_type=jax.ShapeDtypeStruct((NC, N_PAD, D), jnp.float32),
        mesh=_get_mesh(),
        scratch_types=[
            pltpu.VMEM((SUB_A, CH_A), jnp.int32),
            pltpu.VMEM((SUB_A, CH_A), jnp.int32),
            pltpu.VMEM((NBUF, CH_A, D), jnp.float32),
            pltpu.VMEM_SHARED((N_PAD, D), jnp.float32),
            pltpu.SemaphoreType.DMA((NBUF,)),
        ],
    )


@functools.lru_cache(maxsize=None)
def _get_sc_deg():
    return pl.kernel(
        _sc_deg_body,
        out_type=jax.ShapeDtypeStruct((NC, N_PAD, D), jnp.float32),
        mesh=_get_mesh(),
        scratch_types=[
            pltpu.VMEM((SUB_D, CH_D), jnp.int32),
            pltpu.VMEM((CH_D, D), jnp.float32),
            pltpu.VMEM((CH_D, D), jnp.float32),
            pltpu.VMEM_SHARED((N_PAD, D), jnp.float32),
            pltpu.SemaphoreType.DMA,
        ],
    )


RB = 1000


def _tc_layer_body(relu, h_ref, p_ref, dp_ref, w_ref, b_ref, o_ref):
    deg = jnp.maximum(dp_ref[0, :, 0] + dp_ref[1, :, 0], 1.0)
    hn = (p_ref[0] + p_ref[1]) * (1.0 / deg)[:, None]
    x = jnp.concatenate([h_ref[...], hn], axis=1)
    y = jnp.dot(x, w_ref[...], preferred_element_type=jnp.float32) + b_ref[...]
    o_ref[...] = jnp.maximum(y, 0.0) if relu else y


def _tc_layer(h, p, dp, wcat, b, relu, out_dim, interpret=False):
    kin = wcat.shape[0]
    return pl.pallas_call(
        functools.partial(_tc_layer_body, relu),
        grid=(N // RB,),
        in_specs=[
            pl.BlockSpec((RB, D), lambda i: (i, 0)),
            pl.BlockSpec((NC, RB, D), lambda i: (0, i, 0)),
            pl.BlockSpec((NC, RB, 16), lambda i: (0, i, 0)),
            pl.BlockSpec((kin, wcat.shape[1]), lambda i: (0, 0)),
            pl.BlockSpec((1, b.shape[1]), lambda i: (0, 0)),
        ],
        out_specs=pl.BlockSpec((RB, out_dim), lambda i: (i, 0)),
        out_shape=jax.ShapeDtypeStruct((N, out_dim), jnp.float32),
        interpret=interpret,
    )(h, p, dp, wcat, b)


def kernel(edge_index, feats, Wself0, Wneigh0, b0, Wself1, Wneigh1, b1,
           Wself2, Wneigh2, b2):
    src = edge_index[0]
    dst = edge_index[1]
    pad = E_PAD - E
    src_p = jnp.concatenate([src, jnp.zeros((pad,), jnp.int32)])
    dst_p = jnp.concatenate([dst, jnp.full((pad,), N, jnp.int32)])
    srcg = src_p.reshape(TOT_CHUNKS, CH_A)
    dstg = dst_p.reshape(TOT_CHUNKS, CH_A)
    dstg_d = dst_p.reshape(NW, CHUNKS_D, CH_D)

    z128 = jnp.zeros((CH_D, D), jnp.float32)
    z32 = jnp.zeros((CH_A, D), jnp.float32)
    ones128 = jnp.ones((CH_D, D), jnp.float32)

    wcat0 = jnp.concatenate([Wself0, Wneigh0], axis=0)
    wcat1 = jnp.concatenate([Wself1, Wneigh1], axis=0)
    wcat2 = jnp.concatenate([Wself2, Wneigh2], axis=0)

    _sc_deg = _get_sc_deg()
    _sc_agg = _get_sc_agg()
    dpad = _sc_deg(dstg_d, z128, ones128)
    dp = dpad[:, :N, :16]
    p0 = _sc_agg(feats, srcg, dstg, z32)
    h1 = _tc_layer(feats, p0[:, :N], dp, wcat0, b0.reshape(1, -1), True, D)
    p1 = _sc_agg(h1, srcg, dstg, z32)
    h2 = _tc_layer(h1, p1[:, :N], dp, wcat1, b1.reshape(1, -1), True, D)
    p2 = _sc_agg(h2, srcg, dstg, z32)
    logits = _tc_layer(h2, p2[:, :N], dp, wcat2, b2.reshape(1, -1), False,
                       Wself2.shape[1])
    return logits


# speedup vs baseline: 3.6432x; 1.0386x over previous
import functools

import jax
import jax.numpy as jnp
from jax import lax
from jax.experimental import pallas as pl
from jax.experimental.pallas import tpu as pltpu
from jax.experimental.pallas import tpu_sc as plsc

N = 10000
E = 320000
D = 128

NC = 2
NS = 16
NW = NC * NS
CH_A = 32
CHUNKS_A = 320
SUB_A = 32
NBUF = 6
PRE = NBUF - 1
C0 = 544
C1 = 2 * CHUNKS_A - C0
TOT_CHUNKS = NS * (C0 + C1)
CH_D = 128
CHUNKS_D = 80
SUB_D = 16
NSUB_D = CHUNKS_D // SUB_D
E_PAD = NW * CHUNKS_A * CH_A
N_PAD = 10112
RPT = N_PAD // NS

_mesh = None


def _get_mesh():
    global _mesh
    if _mesh is None:
        _mesh = plsc.VectorSubcoreMesh(
            core_axis_name="c", subcore_axis_name="s",
            num_cores=NC, num_subcores=NS,
        )
    return _mesh

def _wchunks(step):
    out, off = [], 0
    while off < RPT:
        out.append((off, min(step, RPT - off)))
        off += step
    return out


def _sc_agg_body(h_hbm, srcg, dstg, z128, agg_out,
                 src_v, dst_v, rows_v, agg_sh, gsems):
    c = lax.axis_index("c")
    s = lax.axis_index("s")
    base = lax.select(c == 0, s * C0, NS * C0 + s * C1)
    trips = lax.select(c == 0, C0 // SUB_A, C1 // SUB_A)

    pltpu.sync_copy(z128, rows_v.at[0])
    for t, sz in _wchunks(CH_A):
        pltpu.sync_copy(rows_v.at[0, pl.ds(0, sz)],
                        agg_sh.at[pl.ds(s * RPT + t, sz)])
    plsc.subcore_barrier()

    def outer(k, carry):
        pltpu.sync_copy(srcg.at[pl.ds(base + k * SUB_A, SUB_A)], src_v)
        pltpu.sync_copy(dstg.at[pl.ds(base + k * SUB_A, SUB_A)], dst_v)

        descs = [None] * NBUF
        for j in range(SUB_A + PRE):
            if j < SUB_A:
                b = j % NBUF
                descs[b] = pltpu.async_copy(
                    h_hbm.at[src_v.at[j]], rows_v.at[b], gsems.at[b])
            if j >= PRE:
                jj = j - PRE
                b = jj % NBUF
                descs[b].wait()
                pltpu.sync_copy(rows_v.at[b], agg_sh.at[dst_v.at[jj]],
                                add=True)
        return carry

    lax.fori_loop(0, trips, outer, 0)
    plsc.subcore_barrier()

    for t, sz in _wchunks(CH_A):
        sl = pl.ds(s * RPT + t, sz)
        pltpu.sync_copy(agg_sh.at[sl], rows_v.at[0, pl.ds(0, sz)])
        pltpu.sync_copy(rows_v.at[0, pl.ds(0, sz)], agg_out.at[c, sl])


def _sc_deg_body(dstg, z128, ones128, deg_out,
                 dst_v, rows_v, ones_v, deg_sh, gsem):
    c = lax.axis_index("c")
    s = lax.axis_index("s")
    wid = s * NC + c

    pltpu.sync_copy(z128, rows_v)
    pltpu.sync_copy(ones128, ones_v)
    for t, sz in _wchunks(CH_D):
        pltpu.sync_copy(rows_v.at[pl.ds(0, sz)],
                        deg_sh.at[pl.ds(s * RPT + t, sz)])
    plsc.subcore_barrier()

    def outer(k, carry):
        pltpu.sync_copy(dstg.at[wid, pl.ds(k * SUB_D, SUB_D)], dst_v)
        for j in range(SUB_D):
            pltpu.sync_copy(ones_v, deg_sh.at[dst_v.at[j]], add=True)
        return carry

    lax.fori_loop(0, NSUB_D, outer, 0)
    plsc.subcore_barrier()

    for t, sz in _wchunks(CH_D):
        sl = pl.ds(s * RPT + t, sz)
        pltpu.sync_copy(deg_sh.at[sl], rows_v.at[pl.ds(0, sz)])
        pltpu.sync_copy(rows_v.at[pl.ds(0, sz)], deg_out.at[c, sl])


@functools.lru_cache(maxsize=None)
def _get_sc_agg():
    return pl.kernel(
        _sc_agg_body,
        out_type=jax.ShapeDtypeStruct((NC, N_PAD, D), jnp.float32),
        mesh=_get_mesh(),
        scratch_types=[
            pltpu.VMEM((SUB_A, CH_A), jnp.int32),
            pltpu.VMEM((SUB_A, CH_A), jnp.int32),
            pltpu.VMEM((NBUF, CH_A, D), jnp.float32),
            pltpu.VMEM_SHARED((N_PAD, D), jnp.float32),
            pltpu.SemaphoreType.DMA((NBUF,)),
        ],
    )


@functools.lru_cache(maxsize=None)
def _get_sc_deg():
    return pl.kernel(
        _sc_deg_body,
        out_type=jax.ShapeDtypeStruct((NC, N_PAD, D), jnp.float32),
        mesh=_get_mesh(),
        scratch_types=[
            pltpu.VMEM((SUB_D, CH_D), jnp.int32),
            pltpu.VMEM((CH_D, D), jnp.float32),
            pltpu.VMEM((CH_D, D), jnp.float32),
            pltpu.VMEM_SHARED((N_PAD, D), jnp.float32),
            pltpu.SemaphoreType.DMA,
        ],
    )


RB = 1000


def _tc_layer_body(relu, h_ref, p_ref, dp_ref, w_ref, b_ref, o_ref):
    deg = jnp.maximum(dp_ref[0, :, 0] + dp_ref[1, :, 0], 1.0)
    hn = (p_ref[0] + p_ref[1]) * (1.0 / deg)[:, None]
    x = jnp.concatenate([h_ref[...], hn], axis=1)
    y = jnp.dot(x, w_ref[...], preferred_element_type=jnp.float32) + b_ref[...]
    o_ref[...] = jnp.maximum(y, 0.0) if relu else y


def _tc_layer(h, p, dp, wcat, b, relu, out_dim, interpret=False):
    kin = wcat.shape[0]
    return pl.pallas_call(
        functools.partial(_tc_layer_body, relu),
        grid=(N // RB,),
        in_specs=[
            pl.BlockSpec((RB, D), lambda i: (i, 0)),
            pl.BlockSpec((NC, RB, D), lambda i: (0, i, 0)),
            pl.BlockSpec((NC, RB, 16), lambda i: (0, i, 0)),
            pl.BlockSpec((kin, wcat.shape[1]), lambda i: (0, 0)),
            pl.BlockSpec((1, b.shape[1]), lambda i: (0, 0)),
        ],
        out_specs=pl.BlockSpec((RB, out_dim), lambda i: (i, 0)),
        out_shape=jax.ShapeDtypeStruct((N, out_dim), jnp.float32),
        interpret=interpret,
    )(h, p, dp, wcat, b)


def kernel(edge_index, feats, Wself0, Wneigh0, b0, Wself1, Wneigh1, b1,
           Wself2, Wneigh2, b2):
    src = edge_index[0]
    dst = edge_index[1]
    pad = E_PAD - E
    src_p = jnp.concatenate([src, jnp.zeros((pad,), jnp.int32)])
    dst_p = jnp.concatenate([dst, jnp.full((pad,), N, jnp.int32)])
    srcg = src_p.reshape(TOT_CHUNKS, CH_A)
    dstg = dst_p.reshape(TOT_CHUNKS, CH_A)
    dstg_d = dst_p.reshape(NW, CHUNKS_D, CH_D)

    z128 = jnp.zeros((CH_D, D), jnp.float32)
    z32 = jnp.zeros((CH_A, D), jnp.float32)
    ones128 = jnp.ones((CH_D, D), jnp.float32)

    wcat0 = jnp.concatenate([Wself0, Wneigh0], axis=0)
    wcat1 = jnp.concatenate([Wself1, Wneigh1], axis=0)
    wcat2 = jnp.concatenate([Wself2, Wneigh2], axis=0)

    _sc_deg = _get_sc_deg()
    _sc_agg = _get_sc_agg()
    dpad = _sc_deg(dstg_d, z128, ones128)
    dp = dpad[:, :N, :16]
    p0 = _sc_agg(feats, srcg, dstg, z32)
    h1 = _tc_layer(feats, p0[:, :N], dp, wcat0, b0.reshape(1, -1), True, D)
    p1 = _sc_agg(h1, srcg, dstg, z32)
    h2 = _tc_layer(h1, p1[:, :N], dp, wcat1, b1.reshape(1, -1), True, D)
    p2 = _sc_agg(h2, srcg, dstg, z32)
    logits = _tc_layer(h2, p2[:, :N], dp, wcat2, b2.reshape(1, -1), False,
                       Wself2.shape[1])
    return logits


# speedup vs baseline: 4.0128x; 1.1015x over previous
import functools

import jax
import jax.numpy as jnp
from jax import lax
from jax.experimental import pallas as pl
from jax.experimental.pallas import tpu as pltpu
from jax.experimental.pallas import tpu_sc as plsc

N = 10000
E = 320000
D = 128

NC = 2
NS = 16
NW = NC * NS
CH_A = 32
CHUNKS_A = 320
SUB_A = 32
NBUF = 6
PRE = NBUF - 1
C0 = 576
C1 = 2 * CHUNKS_A - C0
TOT_CHUNKS = NS * (C0 + C1)
CH_D = 128
CHUNKS_D = 80
SUB_D = 16
NSUB_D = CHUNKS_D // SUB_D
E_PAD = NW * CHUNKS_A * CH_A
N_PAD = 10112
RPT = N_PAD // NS

_mesh = None


def _get_mesh():
    global _mesh
    if _mesh is None:
        _mesh = plsc.VectorSubcoreMesh(
            core_axis_name="c", subcore_axis_name="s",
            num_cores=NC, num_subcores=NS,
        )
    return _mesh

def _wchunks(step):
    out, off = [], 0
    while off < RPT:
        out.append((off, min(step, RPT - off)))
        off += step
    return out


def _sc_agg_body(h_hbm, srcg, dstg, z128, agg_out,
                 src_v, dst_v, rows_v, agg_sh, gsems):
    c = lax.axis_index("c")
    s = lax.axis_index("s")
    base = lax.select(c == 0, s * C0, NS * C0 + s * C1)
    trips = lax.select(c == 0, C0 // SUB_A, C1 // SUB_A)

    pltpu.sync_copy(z128, rows_v.at[0])
    for t, sz in _wchunks(CH_A):
        pltpu.sync_copy(rows_v.at[0, pl.ds(0, sz)],
                        agg_sh.at[pl.ds(s * RPT + t, sz)])
    plsc.subcore_barrier()

    def outer(k, carry):
        pltpu.sync_copy(srcg.at[pl.ds(base + k * SUB_A, SUB_A)], src_v)
        pltpu.sync_copy(dstg.at[pl.ds(base + k * SUB_A, SUB_A)], dst_v)

        descs = [None] * NBUF
        for j in range(SUB_A + PRE):
            if j < SUB_A:
                b = j % NBUF
                descs[b] = pltpu.async_copy(
                    h_hbm.at[src_v.at[j]], rows_v.at[b], gsems.at[b])
            if j >= PRE:
                jj = j - PRE
                b = jj % NBUF
                descs[b].wait()
                pltpu.sync_copy(rows_v.at[b], agg_sh.at[dst_v.at[jj]],
                                add=True)
        return carry

    lax.fori_loop(0, trips, outer, 0)
    plsc.subcore_barrier()

    for t, sz in _wchunks(CH_A):
        sl = pl.ds(s * RPT + t, sz)
        pltpu.sync_copy(agg_sh.at[sl], rows_v.at[0, pl.ds(0, sz)])
        pltpu.sync_copy(rows_v.at[0, pl.ds(0, sz)], agg_out.at[c, sl])


def _sc_deg_body(dstg, z128, ones128, deg_out,
                 dst_v, rows_v, ones_v, deg_sh, gsem):
    c = lax.axis_index("c")
    s = lax.axis_index("s")
    wid = s * NC + c

    pltpu.sync_copy(z128, rows_v)
    pltpu.sync_copy(ones128, ones_v)
    for t, sz in _wchunks(CH_D):
        pltpu.sync_copy(rows_v.at[pl.ds(0, sz)],
                        deg_sh.at[pl.ds(s * RPT + t, sz)])
    plsc.subcore_barrier()

    def outer(k, carry):
        pltpu.sync_copy(dstg.at[wid, pl.ds(k * SUB_D, SUB_D)], dst_v)
        for j in range(SUB_D):
            pltpu.sync_copy(ones_v, deg_sh.at[dst_v.at[j]], add=True)
        return carry

    lax.fori_loop(0, NSUB_D, outer, 0)
    plsc.subcore_barrier()

    for t, sz in _wchunks(CH_D):
        sl = pl.ds(s * RPT + t, sz)
        pltpu.sync_copy(deg_sh.at[sl], rows_v.at[pl.ds(0, sz)])
        pltpu.sync_copy(rows_v.at[pl.ds(0, sz)], deg_out.at[c, sl])


@functools.lru_cache(maxsize=None)
def _get_sc_agg():
    return pl.kernel(
        _sc_agg_body,
        out_type=jax.ShapeDtypeStruct((NC, N_PAD, D), jnp.float32),
        mesh=_get_mesh(),
        scratch_types=[
            pltpu.VMEM((SUB_A, CH_A), jnp.int32),
            pltpu.VMEM((SUB_A, CH_A), jnp.int32),
            pltpu.VMEM((NBUF, CH_A, D), jnp.float32),
            pltpu.VMEM_SHARED((N_PAD, D), jnp.float32),
            pltpu.SemaphoreType.DMA((NBUF,)),
        ],
    )


@functools.lru_cache(maxsize=None)
def _get_sc_deg():
    return pl.kernel(
        _sc_deg_body,
        out_type=jax.ShapeDtypeStruct((NC, N_PAD, D), jnp.float32),
        mesh=_get_mesh(),
        scratch_types=[
            pltpu.VMEM((SUB_D, CH_D), jnp.int32),
            pltpu.VMEM((CH_D, D), jnp.float32),
            pltpu.VMEM((CH_D, D), jnp.float32),
            pltpu.VMEM_SHARED((N_PAD, D), jnp.float32),
            pltpu.SemaphoreType.DMA,
        ],
    )


RB = 1000


def _tc_layer_body(relu, h_ref, p_ref, dp_ref, w_ref, b_ref, o_ref):
    deg = jnp.maximum(dp_ref[0, :, 0] + dp_ref[1, :, 0], 1.0)
    hn = (p_ref[0] + p_ref[1]) * (1.0 / deg)[:, None]
    x = jnp.concatenate([h_ref[...], hn], axis=1)
    y = jnp.dot(x, w_ref[...], preferred_element_type=jnp.float32) + b_ref[...]
    o_ref[...] = jnp.maximum(y, 0.0) if relu else y


def _tc_layer(h, p, dp, wcat, b, relu, out_dim, interpret=False):
    kin = wcat.shape[0]
    return pl.pallas_call(
        functools.partial(_tc_layer_body, relu),
        grid=(N // RB,),
        in_specs=[
            pl.BlockSpec((RB, D), lambda i: (i, 0)),
            pl.BlockSpec((NC, RB, D), lambda i: (0, i, 0)),
            pl.BlockSpec((NC, RB, 16), lambda i: (0, i, 0)),
            pl.BlockSpec((kin, wcat.shape[1]), lambda i: (0, 0)),
            pl.BlockSpec((1, b.shape[1]), lambda i: (0, 0)),
        ],
        out_specs=pl.BlockSpec((RB, out_dim), lambda i: (i, 0)),
        out_shape=jax.ShapeDtypeStruct((N, out_dim), jnp.float32),
        interpret=interpret,
    )(h, p, dp, wcat, b)


def kernel(edge_index, feats, Wself0, Wneigh0, b0, Wself1, Wneigh1, b1,
           Wself2, Wneigh2, b2):
    src = edge_index[0]
    dst = edge_index[1]
    pad = E_PAD - E
    src_p = jnp.concatenate([src, jnp.zeros((pad,), jnp.int32)])
    dst_p = jnp.concatenate([dst, jnp.full((pad,), N, jnp.int32)])
    srcg = src_p.reshape(TOT_CHUNKS, CH_A)
    dstg = dst_p.reshape(TOT_CHUNKS, CH_A)
    dstg_d = dst_p.reshape(NW, CHUNKS_D, CH_D)

    z128 = jnp.zeros((CH_D, D), jnp.float32)
    z32 = jnp.zeros((CH_A, D), jnp.float32)
    ones128 = jnp.ones((CH_D, D), jnp.float32)

    wcat0 = jnp.concatenate([Wself0, Wneigh0], axis=0)
    wcat1 = jnp.concatenate([Wself1, Wneigh1], axis=0)
    wcat2 = jnp.concatenate([Wself2, Wneigh2], axis=0)

    _sc_deg = _get_sc_deg()
    _sc_agg = _get_sc_agg()
    dpad = _sc_deg(dstg_d, z128, ones128)
    dp = dpad[:, :N, :16]
    p0 = _sc_agg(feats, srcg, dstg, z32)
    h1 = _tc_layer(feats, p0[:, :N], dp, wcat0, b0.reshape(1, -1), True, D)
    p1 = _sc_agg(h1, srcg, dstg, z32)
    h2 = _tc_layer(h1, p1[:, :N], dp, wcat1, b1.reshape(1, -1), True, D)
    p2 = _sc_agg(h2, srcg, dstg, z32)
    logits = _tc_layer(h2, p2[:, :N], dp, wcat2, b2.reshape(1, -1), False,
                       Wself2.shape[1])
    return logits


# speedup vs baseline: 4.0727x; 1.0149x over previous
import functools

import jax
import jax.numpy as jnp
from jax import lax
from jax.experimental import pallas as pl
from jax.experimental.pallas import tpu as pltpu
from jax.experimental.pallas import tpu_sc as plsc

N = 10000
E = 320000
D = 128

NC = 2
NS = 16
NW = NC * NS
CH_A = 32
CHUNKS_A = 320
SUB_A = 32
NBUF = 6
PRE = NBUF - 1
C0 = 608
C1 = 2 * CHUNKS_A - C0
TOT_CHUNKS = NS * (C0 + C1)
CH_D = 128
CHUNKS_D = 80
SUB_D = 16
NSUB_D = CHUNKS_D // SUB_D
E_PAD = NW * CHUNKS_A * CH_A
N_PAD = 10112
RPT = N_PAD // NS

_mesh = None


def _get_mesh():
    global _mesh
    if _mesh is None:
        _mesh = plsc.VectorSubcoreMesh(
            core_axis_name="c", subcore_axis_name="s",
            num_cores=NC, num_subcores=NS,
        )
    return _mesh

def _wchunks(step):
    out, off = [], 0
    while off < RPT:
        out.append((off, min(step, RPT - off)))
        off += step
    return out


def _sc_agg_body(h_hbm, srcg, dstg, z128, agg_out,
                 src_v, dst_v, rows_v, agg_sh, gsems):
    c = lax.axis_index("c")
    s = lax.axis_index("s")
    base = lax.select(c == 0, s * C0, NS * C0 + s * C1)
    trips = lax.select(c == 0, C0 // SUB_A, C1 // SUB_A)

    pltpu.sync_copy(z128, rows_v.at[0])
    for t, sz in _wchunks(CH_A):
        pltpu.sync_copy(rows_v.at[0, pl.ds(0, sz)],
                        agg_sh.at[pl.ds(s * RPT + t, sz)])
    plsc.subcore_barrier()

    def outer(k, carry):
        pltpu.sync_copy(srcg.at[pl.ds(base + k * SUB_A, SUB_A)], src_v)
        pltpu.sync_copy(dstg.at[pl.ds(base + k * SUB_A, SUB_A)], dst_v)

        descs = [None] * NBUF
        for j in range(SUB_A + PRE):
            if j < SUB_A:
                b = j % NBUF
                descs[b] = pltpu.async_copy(
                    h_hbm.at[src_v.at[j]], rows_v.at[b], gsems.at[b])
            if j >= PRE:
                jj = j - PRE
                b = jj % NBUF
                descs[b].wait()
                pltpu.sync_copy(rows_v.at[b], agg_sh.at[dst_v.at[jj]],
                                add=True)
        return carry

    lax.fori_loop(0, trips, outer, 0)
    plsc.subcore_barrier()

    for t, sz in _wchunks(CH_A):
        sl = pl.ds(s * RPT + t, sz)
        pltpu.sync_copy(agg_sh.at[sl], rows_v.at[0, pl.ds(0, sz)])
        pltpu.sync_copy(rows_v.at[0, pl.ds(0, sz)], agg_out.at[c, sl])


def _sc_deg_body(dstg, z128, ones128, deg_out,
                 dst_v, rows_v, ones_v, deg_sh, gsem):
    c = lax.axis_index("c")
    s = lax.axis_index("s")
    wid = s * NC + c

    pltpu.sync_copy(z128, rows_v)
    pltpu.sync_copy(ones128, ones_v)
    for t, sz in _wchunks(CH_D):
        pltpu.sync_copy(rows_v.at[pl.ds(0, sz)],
                        deg_sh.at[pl.ds(s * RPT + t, sz)])
    plsc.subcore_barrier()

    def outer(k, carry):
        pltpu.sync_copy(dstg.at[wid, pl.ds(k * SUB_D, SUB_D)], dst_v)
        for j in range(SUB_D):
            pltpu.sync_copy(ones_v, deg_sh.at[dst_v.at[j]], add=True)
        return carry

    lax.fori_loop(0, NSUB_D, outer, 0)
    plsc.subcore_barrier()

    for t, sz in _wchunks(CH_D):
        sl = pl.ds(s * RPT + t, sz)
        pltpu.sync_copy(deg_sh.at[sl], rows_v.at[pl.ds(0, sz)])
        pltpu.sync_copy(rows_v.at[pl.ds(0, sz)], deg_out.at[c, sl])


@functools.lru_cache(maxsize=None)
def _get_sc_agg():
    return pl.kernel(
        _sc_agg_body,
        out_type=jax.ShapeDtypeStruct((NC, N_PAD, D), jnp.float32),
        mesh=_get_mesh(),
        scratch_types=[
            pltpu.VMEM((SUB_A, CH_A), jnp.int32),
            pltpu.VMEM((SUB_A, CH_A), jnp.int32),
            pltpu.VMEM((NBUF, CH_A, D), jnp.float32),
            pltpu.VMEM_SHARED((N_PAD, D), jnp.float32),
            pltpu.SemaphoreType.DMA((NBUF,)),
        ],
    )


@functools.lru_cache(maxsize=None)
def _get_sc_deg():
    return pl.kernel(
        _sc_deg_body,
        out_type=jax.ShapeDtypeStruct((NC, N_PAD, D), jnp.float32),
        mesh=_get_mesh(),
        scratch_types=[
            pltpu.VMEM((SUB_D, CH_D), jnp.int32),
            pltpu.VMEM((CH_D, D), jnp.float32),
            pltpu.VMEM((CH_D, D), jnp.float32),
            pltpu.VMEM_SHARED((N_PAD, D), jnp.float32),
            pltpu.SemaphoreType.DMA,
        ],
    )


RB = 1000


def _tc_layer_body(relu, h_ref, p_ref, dp_ref, w_ref, b_ref, o_ref):
    deg = jnp.maximum(dp_ref[0, :, 0] + dp_ref[1, :, 0], 1.0)
    hn = (p_ref[0] + p_ref[1]) * (1.0 / deg)[:, None]
    x = jnp.concatenate([h_ref[...], hn], axis=1)
    y = jnp.dot(x, w_ref[...], preferred_element_type=jnp.float32) + b_ref[...]
    o_ref[...] = jnp.maximum(y, 0.0) if relu else y


def _tc_layer(h, p, dp, wcat, b, relu, out_dim, interpret=False):
    kin = wcat.shape[0]
    return pl.pallas_call(
        functools.partial(_tc_layer_body, relu),
        grid=(N // RB,),
        in_specs=[
            pl.BlockSpec((RB, D), lambda i: (i, 0)),
            pl.BlockSpec((NC, RB, D), lambda i: (0, i, 0)),
            pl.BlockSpec((NC, RB, 16), lambda i: (0, i, 0)),
            pl.BlockSpec((kin, wcat.shape[1]), lambda i: (0, 0)),
            pl.BlockSpec((1, b.shape[1]), lambda i: (0, 0)),
        ],
        out_specs=pl.BlockSpec((RB, out_dim), lambda i: (i, 0)),
        out_shape=jax.ShapeDtypeStruct((N, out_dim), jnp.float32),
        interpret=interpret,
    )(h, p, dp, wcat, b)


def kernel(edge_index, feats, Wself0, Wneigh0, b0, Wself1, Wneigh1, b1,
           Wself2, Wneigh2, b2):
    src = edge_index[0]
    dst = edge_index[1]
    pad = E_PAD - E
    src_p = jnp.concatenate([src, jnp.zeros((pad,), jnp.int32)])
    dst_p = jnp.concatenate([dst, jnp.full((pad,), N, jnp.int32)])
    srcg = src_p.reshape(TOT_CHUNKS, CH_A)
    dstg = dst_p.reshape(TOT_CHUNKS, CH_A)
    dstg_d = dst_p.reshape(NW, CHUNKS_D, CH_D)

    z128 = jnp.zeros((CH_D, D), jnp.float32)
    z32 = jnp.zeros((CH_A, D), jnp.float32)
    ones128 = jnp.ones((CH_D, D), jnp.float32)

    wcat0 = jnp.concatenate([Wself0, Wneigh0], axis=0)
    wcat1 = jnp.concatenate([Wself1, Wneigh1], axis=0)
    wcat2 = jnp.concatenate([Wself2, Wneigh2], axis=0)

    _sc_deg = _get_sc_deg()
    _sc_agg = _get_sc_agg()
    dpad = _sc_deg(dstg_d, z128, ones128)
    dp = dpad[:, :N, :16]
    p0 = _sc_agg(feats, srcg, dstg, z32)
    h1 = _tc_layer(feats, p0[:, :N], dp, wcat0, b0.reshape(1, -1), True, D)
    p1 = _sc_agg(h1, srcg, dstg, z32)
    h2 = _tc_layer(h1, p1[:, :N], dp, wcat1, b1.reshape(1, -1), True, D)
    p2 = _sc_agg(h2, srcg, dstg, z32)
    logits = _tc_layer(h2, p2[:, :N], dp, wcat2, b2.reshape(1, -1), False,
                       Wself2.shape[1])
    return logits
